# baseline (speedup 1.0000x reference)
import functools

import jax
import jax.numpy as jnp
from jax import lax
from jax.experimental import pallas as pl
from jax.experimental.pallas import tpu as pltpu
from jax.experimental.pallas import tpu_sc as plsc

F32 = jnp.float32

_N = 10000
_E = 320000
_DC = 16
_NG = 64

_NC = 2
_NS = 16
_NW = _NC * _NS
_EPW = _E // _NW
_CH = 80
_NJ = _EPW // _CH


def _proj_body(x_ref, w_ref, b_ref, o_ref):
    o_ref[...] = jax.nn.relu(
        jnp.dot(x_ref[...], w_ref[...], preferred_element_type=F32) + b_ref[...])


def _projection(x, w0, b0):
    nblk = 5
    bm = _N // nblk
    return pl.pallas_call(
        _proj_body,
        grid=(nblk,),
        in_specs=[
            pl.BlockSpec((bm, x.shape[1]), lambda i: (i, 0)),
            pl.BlockSpec(w0.shape, lambda i: (0, 0)),
            pl.BlockSpec((1, _DC), lambda i: (0, 0)),
        ],
        out_specs=pl.BlockSpec((bm, _DC), lambda i: (i, 0)),
        out_shape=jax.ShapeDtypeStruct((_N, _DC), F32),
    )(x, w0, b0.reshape(1, -1))


def _msg_body(ea_ref, g_ref, a1_ref, bn1_ref, a2_ref, bn2_ref, o_ref):
    eye = (lax.broadcasted_iota(jnp.int32, (_DC, _DC), 0) ==
           lax.broadcasted_iota(jnp.int32, (_DC, _DC), 1)).astype(F32)
    dn0 = (((0,), (1,)), ((), ()))
    dn1 = (((0,), (0,)), ((), ()))
    rh_t = jax.nn.relu(
        lax.dot_general(a1_ref[...], ea_ref[...], dn1,
                        preferred_element_type=F32) + bn1_ref[...])
    ew_t = (lax.dot_general(a2_ref[...].astype(jnp.bfloat16),
                            rh_t.astype(jnp.bfloat16), dn1,
                            preferred_element_type=F32) + bn2_ref[...])
    g_t = lax.dot_general(eye, g_ref[...], dn0, preferred_element_type=F32)
    acc = g_t[0:1, :] * ew_t[0:_DC, :]
    for i in range(1, _DC):
        acc = acc + g_t[i:i + 1, :] * ew_t[i * _DC:(i + 1) * _DC, :]
    o_ref[...] = lax.dot_general(acc, eye, dn1, preferred_element_type=F32)


def _messages(edge_attr_t, g, a1, bn1, a2, bn2):
    be = 2560
    nblk = _E // be
    return pl.pallas_call(
        _msg_body,
        grid=(nblk,),
        in_specs=[
            pl.BlockSpec((edge_attr_t.shape[0], be), lambda i: (0, i)),
            pl.BlockSpec((be, _DC), lambda i: (i, 0)),
            pl.BlockSpec(a1.shape, lambda i: (0, 0)),
            pl.BlockSpec((a1.shape[1], 1), lambda i: (0, 0)),
            pl.BlockSpec(a2.shape, lambda i: (0, 0)),
            pl.BlockSpec((a2.shape[1], 1), lambda i: (0, 0)),
        ],
        out_specs=pl.BlockSpec((be, _DC), lambda i: (i, 0)),
        out_shape=jax.ShapeDtypeStruct((_E, _DC), F32),
    )(edge_attr_t, g, a1, bn1.reshape(-1, 1), a2, bn2.reshape(-1, 1))


@functools.cache
def _sc_mesh():
    return plsc.VectorSubcoreMesh(core_axis_name="c", subcore_axis_name="s",
                                  num_cores=_NC, num_subcores=_NS)


_GCH = 2000
_GNO = _EPW // _GCH
_GIN = _GCH // _CH


def _sc_gather_body(table_hbm, src_hbm, out_hbm, idx_v, rows_a, rows_b,
                    table_sh, gsem, osem_a, osem_b):
    c = lax.axis_index("c")
    s = lax.axis_index("s")
    wid = s * _NC + c
    pltpu.sync_copy(table_hbm.at[pl.ds(s * _RPT, _RPT)],
                    table_sh.at[pl.ds(s * _RPT, _RPT)])
    pltpu.sync_copy(src_hbm.at[pl.ds(wid * _NJ, _NJ)], idx_v)
    plsc.subcore_barrier()
    bufs = ((rows_a, osem_a), (rows_b, osem_b))
    for jo in range(_GNO):
        rows_v, osem = bufs[jo % 2]
        if jo >= 2:
            pltpu.make_async_copy(rows_v, out_hbm.at[pl.ds(0, _GCH)],
                                  osem).wait()

        def fire(k, carry):
            pltpu.async_copy(table_sh.at[idx_v.at[jo * _GIN + k]],
                             rows_v.at[pl.ds(k * _CH, _CH)], gsem)
            return carry

        lax.fori_loop(0, _GIN, fire, 0)

        def drain(k, carry):
            pltpu.make_async_copy(table_sh.at[idx_v.at[jo * _GIN + k]],
                                  rows_v.at[pl.ds(k * _CH, _CH)],
                                  gsem).wait()
            return carry

        lax.fori_loop(0, _GIN, drain, 0)
        pltpu.async_copy(rows_v,
                         out_hbm.at[pl.ds(wid * _EPW + jo * _GCH, _GCH)],
                         osem)
    for jo in (_GNO - 2, _GNO - 1):
        rows_v, osem = bufs[jo % 2]
        pltpu.make_async_copy(rows_v, out_hbm.at[pl.ds(0, _GCH)], osem).wait()


@functools.cache
def _sc_gather():
    return pl.kernel(
        _sc_gather_body,
        out_type=jax.ShapeDtypeStruct((_E, _DC), F32),
        mesh=_sc_mesh(),
        compiler_params=pltpu.CompilerParams(use_tc_tiling_on_sc=False),
        scratch_types=[
            pltpu.VMEM((_NJ, _CH), jnp.int32),
            pltpu.VMEM((_GCH, _DC), F32),
            pltpu.VMEM((_GCH, _DC), F32),
            pltpu.VMEM_SHARED((_N, _DC), F32),
            pltpu.SemaphoreType.DMA,
            pltpu.SemaphoreType.DMA,
            pltpu.SemaphoreType.DMA,
        ],
    )


_RPT = _N // _NS


def _make_scatter_body(with_cnt):
    def body_fn(msg_hbm, dst_hbm, pagg_hbm, *rest):
        if with_cnt:
            (pcnt_hbm, idx_v, msg_a, msg_b, ones_v, buf_v, agg_sh, cnt_sh,
             lsem_a, lsem_b, ssem, csem) = rest
        else:
            (idx_v, msg_a, msg_b, buf_v, agg_sh,
             lsem_a, lsem_b, ssem) = rest
        c = lax.axis_index("c")
        s = lax.axis_index("s")
        wid = s * _NC + c

        def fill_zeros(i, carry):
            buf_v[i, :] = jnp.zeros((_DC,), F32)
            return carry

        lax.fori_loop(0, _RPT, fill_zeros, 0)
        pltpu.sync_copy(buf_v, agg_sh.at[pl.ds(s * _RPT, _RPT)])
        if with_cnt:
            pltpu.sync_copy(buf_v, cnt_sh.at[pl.ds(s * _RPT, _RPT)])

            def fill_ones(i, carry):
                ones_v[i, :] = jnp.ones((_DC,), F32)
                return carry

            lax.fori_loop(0, _CH, fill_ones, 0)
        pltpu.sync_copy(dst_hbm.at[pl.ds(wid * _NJ, _NJ)], idx_v)
        plsc.subcore_barrier()

        bufs = ((msg_a, lsem_a), (msg_b, lsem_b))
        pltpu.async_copy(msg_hbm.at[pl.ds(wid * _EPW, _GCH)], msg_a, lsem_a)
        for jo in range(_GNO):
            msg_v, lsem = bufs[jo % 2]
            pltpu.make_async_copy(msg_hbm.at[pl.ds(0, _GCH)], msg_v,
                                  lsem).wait()
            if jo + 1 < _GNO:
                nxt_v, nxt_sem = bufs[(jo + 1) % 2]
                pltpu.async_copy(
                    msg_hbm.at[pl.ds(wid * _EPW + (jo + 1) * _GCH, _GCH)],
                    nxt_v, nxt_sem)

            def fire(k, carry):
                pltpu.async_copy(msg_v.at[pl.ds(k * _CH, _CH)],
                                 agg_sh.at[idx_v.at[jo * _GIN + k]], ssem,
                                 add=True)
                if with_cnt:
                    pltpu.async_copy(ones_v,
                                     cnt_sh.at[idx_v.at[jo * _GIN + k]],
                                     csem, add=True)
                return carry

            lax.fori_loop(0, _GIN, fire, 0)

            def drain(k, carry):
                pltpu.make_async_copy(msg_v.at[pl.ds(k * _CH, _CH)],
                                      agg_sh.at[idx_v.at[jo * _GIN + k]],
                                      ssem).wait()
                if with_cnt:
                    pltpu.make_async_copy(ones_v,
                                          cnt_sh.at[idx_v.at[jo * _GIN + k]],
                                          csem).wait()
                return carry

            lax.fori_loop(0, _GIN, drain, 0)
        plsc.subcore_barrier()

        pltpu.sync_copy(agg_sh.at[pl.ds(s * _RPT, _RPT)], buf_v)
        pltpu.sync_copy(buf_v, pagg_hbm.at[c].at[pl.ds(s * _RPT, _RPT)])
        if with_cnt:
            pltpu.sync_copy(cnt_sh.at[pl.ds(s * _RPT, _RPT)], buf_v)
            pltpu.sync_copy(buf_v, pcnt_hbm.at[c].at[pl.ds(s * _RPT, _RPT)])

    return body_fn


@functools.cache
def _sc_scatter(with_cnt):
    out_type = [jax.ShapeDtypeStruct((_NC, _N, _DC), F32)]
    scratch = [
        pltpu.VMEM((_NJ, _CH), jnp.int32),
        pltpu.VMEM((_GCH, _DC), F32),
        pltpu.VMEM((_GCH, _DC), F32),
    ]
    if with_cnt:
        out_type.append(jax.ShapeDtypeStruct((_NC, _N, _DC), F32))
        scratch.append(pltpu.VMEM((_CH, _DC), F32))
    scratch.append(pltpu.VMEM((_RPT, _DC), F32))
    scratch.append(pltpu.VMEM_SHARED((_N, _DC), F32))
    if with_cnt:
        scratch.append(pltpu.VMEM_SHARED((_N, _DC), F32))
    scratch += [pltpu.SemaphoreType.DMA, pltpu.SemaphoreType.DMA,
                pltpu.SemaphoreType.DMA]
    if with_cnt:
        scratch.append(pltpu.SemaphoreType.DMA)
    return pl.kernel(
        _make_scatter_body(with_cnt),
        out_type=tuple(out_type) if with_cnt else out_type[0],
        mesh=_sc_mesh(),
        compiler_params=pltpu.CompilerParams(use_tc_tiling_on_sc=False),
        scratch_types=scratch,
    )


def _gru_core(pa0, pa1, pc0, pc1, h, root, convb, wihT, bih, whhT, bhh):
    agg = pa0 + pa1
    cnt = pc0 + pc1
    mean = agg / jnp.maximum(cnt, 1.0)
    m = jax.nn.relu(
        mean + jnp.dot(h, root, preferred_element_type=F32) + convb)

    def gate(k):
        lo = k * _DC
        gi = (jnp.dot(m, wihT[:, lo:lo + _DC], preferred_element_type=F32)
              + bih[:, lo:lo + _DC])
        gh = (jnp.dot(h, whhT[:, lo:lo + _DC], preferred_element_type=F32)
              + bhh[:, lo:lo + _DC])
        return gi, gh

    i_r, h_r = gate(0)
    i_z, h_z = gate(1)
    i_n, h_n = gate(2)
    r = jax.nn.sigmoid(i_r + h_r)
    z = jax.nn.sigmoid(i_z + h_z)
    n = jnp.tanh(i_n + r * h_n)
    return (1.0 - z) * n + z * h


def _update_body(pa0_ref, pa1_ref, pc0_ref, pc1_ref, h_ref, root_ref,
                 convb_ref, wihT_ref, bih_ref, whhT_ref, bhh_ref, o_ref):
    o_ref[...] = _gru_core(
        pa0_ref[...], pa1_ref[...], pc0_ref[...], pc1_ref[...], h_ref[...],
        root_ref[...], convb_ref[...], wihT_ref[...], bih_ref[...],
        whhT_ref[...], bhh_ref[...])


def _update_final_body(pa0_ref, pa1_ref, pc0_ref, pc1_ref, h_ref, root_ref,
                       convb_ref, wihT_ref, bih_ref, whhT_ref, bhh_ref,
                       batch_ref, gattr_ref, w1a_ref, w1b_ref, bl1_ref,
                       w2_ref, bl2_ref, o_ref):
    h = _gru_core(
        pa0_ref[...], pa1_ref[...], pc0_ref[...], pc1_ref[...], h_ref[...],
        root_ref[...], convb_ref[...], wihT_ref[...], bih_ref[...],
        whhT_ref[...], bhh_ref[...])
    oh = (batch_ref[...] ==
          lax.broadcasted_iota(jnp.int32, (h.shape[0], _NG), 1)).astype(F32)
    gb = jnp.dot(oh, gattr_ref[...], preferred_element_type=F32)
    t = jax.nn.relu(
        jnp.dot(h, w1a_ref[...], preferred_element_type=F32)
        + jnp.dot(gb, w1b_ref[...], preferred_element_type=F32)
        + bl1_ref[...])
    o_ref[...] = (jnp.dot(t, w2_ref[...], preferred_element_type=F32)
                  + bl2_ref[...])


def _full(shape):
    return pl.BlockSpec(shape, lambda: (0,) * len(shape))


def _update(pagg, pcnt, h, root, convb, wihT, bih, whhT, bhh):
    args = (pagg[0], pagg[1], pcnt[0], pcnt[1], h, root,
            convb.reshape(1, -1), wihT, bih.reshape(1, -1), whhT,
            bhh.reshape(1, -1))
    return pl.pallas_call(
        _update_body,
        in_specs=[_full(a.shape) for a in args],
        out_specs=_full((_N, _DC)),
        out_shape=jax.ShapeDtypeStruct((_N, _DC), F32),
    )(*args)


def _update_final(pagg, pcnt, h, root, convb, wihT, bih, whhT, bhh,
                  batch, gattr, w1, bl1, w2, bl2):
    args = (pagg[0], pagg[1], pcnt[0], pcnt[1], h, root,
            convb.reshape(1, -1), wihT, bih.reshape(1, -1), whhT,
            bhh.reshape(1, -1), batch.reshape(-1, 1), gattr,
            w1[:_DC], w1[_DC:], bl1.reshape(1, -1), w2, bl2.reshape(1, -1))
    return pl.pallas_call(
        _update_final_body,
        in_specs=[_full(a.shape) for a in args],
        out_specs=_full((_N, w2.shape[1])),
        out_shape=jax.ShapeDtypeStruct((_N, w2.shape[1]), F32),
    )(*args)


def kernel(x, edge_index, edge_attr, graph_attr, batch, W0, b0, A1, bn1, A2,
           bn2, root, convb, Wih, bih, Whh, bhh, W1, bl1, W2, bl2):
    src2d = edge_index[0].reshape(_E // _CH, _CH)
    dst2d = edge_index[1].reshape(_E // _CH, _CH)
    wihT = Wih.T
    whhT = Whh.T
    ea_t = edge_attr.T

    h = _projection(x, W0, b0)

    g = _sc_gather()(h, src2d)
    msg = _messages(ea_t, g, A1, bn1, A2, bn2)
    pagg, pcnt = _sc_scatter(True)(msg, dst2d)
    h = _update(pagg, pcnt, h, root, convb, wihT, bih, whhT, bhh)

    g = _sc_gather()(h, src2d)
    msg = _messages(ea_t, g, A1, bn1, A2, bn2)
    pagg = _sc_scatter(False)(msg, dst2d)
    return _update_final(pagg, pcnt, h, root, convb, wihT, bih, whhT,
                         bhh, batch, graph_attr, W1, bl1, W2, bl2)


# speedup vs baseline: 1.0008x; 1.0008x over previous
import functools

import jax
import jax.numpy as jnp
from jax import lax
from jax.experimental import pallas as pl
from jax.experimental.pallas import tpu as pltpu
from jax.experimental.pallas import tpu_sc as plsc

F32 = jnp.float32

_N = 10000
_E = 320000
_DC = 16
_NG = 64

_NC = 2
_NS = 16
_NW = _NC * _NS
_EPW = _E // _NW
_CH = 80
_NJ = _EPW // _CH


def _proj_body(x_ref, w_ref, b_ref, o_ref):
    o_ref[...] = jax.nn.relu(
        jnp.dot(x_ref[...], w_ref[...], preferred_element_type=F32) + b_ref[...])


def _projection(x, w0, b0):
    nblk = 5
    bm = _N // nblk
    return pl.pallas_call(
        _proj_body,
        grid=(nblk,),
        in_specs=[
            pl.BlockSpec((bm, x.shape[1]), lambda i: (i, 0)),
            pl.BlockSpec(w0.shape, lambda i: (0, 0)),
            pl.BlockSpec((1, _DC), lambda i: (0, 0)),
        ],
        out_specs=pl.BlockSpec((bm, _DC), lambda i: (i, 0)),
        out_shape=jax.ShapeDtypeStruct((_N, _DC), F32),
    )(x, w0, b0.reshape(1, -1))


def _msg_body(ea_ref, g_ref, a1_ref, bn1_ref, a2_ref, bn2_ref, o_ref):
    eye = (lax.broadcasted_iota(jnp.int32, (_DC, _DC), 0) ==
           lax.broadcasted_iota(jnp.int32, (_DC, _DC), 1)).astype(F32)
    dn0 = (((0,), (1,)), ((), ()))
    dn1 = (((0,), (0,)), ((), ()))
    rh_t = jax.nn.relu(
        lax.dot_general(a1_ref[...], ea_ref[...], dn1,
                        preferred_element_type=F32) + bn1_ref[...])
    ew_t = (lax.dot_general(a2_ref[...], rh_t, dn1,
                            preferred_element_type=F32) + bn2_ref[...])
    g_t = lax.dot_general(eye, g_ref[...], dn0, preferred_element_type=F32)
    acc = g_t[0:1, :] * ew_t[0:_DC, :]
    for i in range(1, _DC):
        acc = acc + g_t[i:i + 1, :] * ew_t[i * _DC:(i + 1) * _DC, :]
    o_ref[...] = lax.dot_general(acc, eye, dn1, preferred_element_type=F32)


def _messages(edge_attr_t, g, a1, bn1, a2, bn2):
    be = 2560
    nblk = _E // be
    return pl.pallas_call(
        _msg_body,
        grid=(nblk,),
        in_specs=[
            pl.BlockSpec((edge_attr_t.shape[0], be), lambda i: (0, i)),
            pl.BlockSpec((be, _DC), lambda i: (i, 0)),
            pl.BlockSpec(a1.shape, lambda i: (0, 0)),
            pl.BlockSpec((a1.shape[1], 1), lambda i: (0, 0)),
            pl.BlockSpec(a2.shape, lambda i: (0, 0)),
            pl.BlockSpec((a2.shape[1], 1), lambda i: (0, 0)),
        ],
        out_specs=pl.BlockSpec((be, _DC), lambda i: (i, 0)),
        out_shape=jax.ShapeDtypeStruct((_E, _DC), F32),
    )(edge_attr_t, g, a1, bn1.reshape(-1, 1), a2, bn2.reshape(-1, 1))


@functools.cache
def _sc_mesh():
    return plsc.VectorSubcoreMesh(core_axis_name="c", subcore_axis_name="s",
                                  num_cores=_NC, num_subcores=_NS)


_GCH = 2000
_GNO = _EPW // _GCH
_GIN = _GCH // _CH


def _sc_gather_body(table_hbm, src_hbm, out_hbm, idx_v, rows_a, rows_b,
                    table_sh, gsem, osem_a, osem_b):
    c = lax.axis_index("c")
    s = lax.axis_index("s")
    wid = s * _NC + c
    pltpu.sync_copy(table_hbm.at[pl.ds(s * _RPT, _RPT)],
                    table_sh.at[pl.ds(s * _RPT, _RPT)])
    pltpu.sync_copy(src_hbm.at[pl.ds(wid * _NJ, _NJ)], idx_v)
    plsc.subcore_barrier()
    bufs = ((rows_a, osem_a), (rows_b, osem_b))
    for jo in range(_GNO):
        rows_v, osem = bufs[jo % 2]
        if jo >= 2:
            pltpu.make_async_copy(rows_v, out_hbm.at[pl.ds(0, _GCH)],
                                  osem).wait()

        def fire(k, carry):
            pltpu.async_copy(table_sh.at[idx_v.at[jo * _GIN + k]],
                             rows_v.at[pl.ds(k * _CH, _CH)], gsem)
            return carry

        lax.fori_loop(0, _GIN, fire, 0)

        def drain(k, carry):
            pltpu.make_async_copy(table_sh.at[idx_v.at[jo * _GIN + k]],
                                  rows_v.at[pl.ds(k * _CH, _CH)],
                                  gsem).wait()
            return carry

        lax.fori_loop(0, _GIN, drain, 0)
        pltpu.async_copy(rows_v,
                         out_hbm.at[pl.ds(wid * _EPW + jo * _GCH, _GCH)],
                         osem)
    for jo in (_GNO - 2, _GNO - 1):
        rows_v, osem = bufs[jo % 2]
        pltpu.make_async_copy(rows_v, out_hbm.at[pl.ds(0, _GCH)], osem).wait()


@functools.cache
def _sc_gather():
    return pl.kernel(
        _sc_gather_body,
        out_type=jax.ShapeDtypeStruct((_E, _DC), F32),
        mesh=_sc_mesh(),
        compiler_params=pltpu.CompilerParams(use_tc_tiling_on_sc=False),
        scratch_types=[
            pltpu.VMEM((_NJ, _CH), jnp.int32),
            pltpu.VMEM((_GCH, _DC), F32),
            pltpu.VMEM((_GCH, _DC), F32),
            pltpu.VMEM_SHARED((_N, _DC), F32),
            pltpu.SemaphoreType.DMA,
            pltpu.SemaphoreType.DMA,
            pltpu.SemaphoreType.DMA,
        ],
    )


_RPT = _N // _NS


def _make_scatter_body(with_cnt):
    def body_fn(msg_hbm, dst_hbm, pagg_hbm, *rest):
        if with_cnt:
            (pcnt_hbm, idx_v, msg_a, msg_b, ones_v, buf_v, agg_sh, cnt_sh,
             lsem_a, lsem_b, ssem, csem) = rest
        else:
            (idx_v, msg_a, msg_b, buf_v, agg_sh,
             lsem_a, lsem_b, ssem) = rest
        c = lax.axis_index("c")
        s = lax.axis_index("s")
        wid = s * _NC + c

        def fill_zeros(i, carry):
            buf_v[i, :] = jnp.zeros((_DC,), F32)
            return carry

        lax.fori_loop(0, _RPT, fill_zeros, 0)
        pltpu.sync_copy(buf_v, agg_sh.at[pl.ds(s * _RPT, _RPT)])
        if with_cnt:
            pltpu.sync_copy(buf_v, cnt_sh.at[pl.ds(s * _RPT, _RPT)])

            def fill_ones(i, carry):
                ones_v[i, :] = jnp.ones((_DC,), F32)
                return carry

            lax.fori_loop(0, _CH, fill_ones, 0)
        pltpu.sync_copy(dst_hbm.at[pl.ds(wid * _NJ, _NJ)], idx_v)
        plsc.subcore_barrier()

        bufs = ((msg_a, lsem_a), (msg_b, lsem_b))
        pltpu.async_copy(msg_hbm.at[pl.ds(wid * _EPW, _GCH)], msg_a, lsem_a)
        for jo in range(_GNO):
            msg_v, lsem = bufs[jo % 2]
            pltpu.make_async_copy(msg_hbm.at[pl.ds(0, _GCH)], msg_v,
                                  lsem).wait()
            if jo + 1 < _GNO:
                nxt_v, nxt_sem = bufs[(jo + 1) % 2]
                pltpu.async_copy(
                    msg_hbm.at[pl.ds(wid * _EPW + (jo + 1) * _GCH, _GCH)],
                    nxt_v, nxt_sem)

            def fire(k, carry):
                pltpu.async_copy(msg_v.at[pl.ds(k * _CH, _CH)],
                                 agg_sh.at[idx_v.at[jo * _GIN + k]], ssem,
                                 add=True)
                if with_cnt:
                    pltpu.async_copy(ones_v,
                                     cnt_sh.at[idx_v.at[jo * _GIN + k]],
                                     csem, add=True)
                return carry

            lax.fori_loop(0, _GIN, fire, 0)

            def drain(k, carry):
                pltpu.make_async_copy(msg_v.at[pl.ds(k * _CH, _CH)],
                                      agg_sh.at[idx_v.at[jo * _GIN + k]],
                                      ssem).wait()
                if with_cnt:
                    pltpu.make_async_copy(ones_v,
                                          cnt_sh.at[idx_v.at[jo * _GIN + k]],
                                          csem).wait()
                return carry

            lax.fori_loop(0, _GIN, drain, 0)
        plsc.subcore_barrier()

        pltpu.sync_copy(agg_sh.at[pl.ds(s * _RPT, _RPT)], buf_v)
        pltpu.sync_copy(buf_v, pagg_hbm.at[c].at[pl.ds(s * _RPT, _RPT)])
        if with_cnt:
            pltpu.sync_copy(cnt_sh.at[pl.ds(s * _RPT, _RPT)], buf_v)
            pltpu.sync_copy(buf_v, pcnt_hbm.at[c].at[pl.ds(s * _RPT, _RPT)])

    return body_fn


@functools.cache
def _sc_scatter(with_cnt):
    out_type = [jax.ShapeDtypeStruct((_NC, _N, _DC), F32)]
    scratch = [
        pltpu.VMEM((_NJ, _CH), jnp.int32),
        pltpu.VMEM((_GCH, _DC), F32),
        pltpu.VMEM((_GCH, _DC), F32),
    ]
    if with_cnt:
        out_type.append(jax.ShapeDtypeStruct((_NC, _N, _DC), F32))
        scratch.append(pltpu.VMEM((_CH, _DC), F32))
    scratch.append(pltpu.VMEM((_RPT, _DC), F32))
    scratch.append(pltpu.VMEM_SHARED((_N, _DC), F32))
    if with_cnt:
        scratch.append(pltpu.VMEM_SHARED((_N, _DC), F32))
    scratch += [pltpu.SemaphoreType.DMA, pltpu.SemaphoreType.DMA,
                pltpu.SemaphoreType.DMA]
    if with_cnt:
        scratch.append(pltpu.SemaphoreType.DMA)
    return pl.kernel(
        _make_scatter_body(with_cnt),
        out_type=tuple(out_type) if with_cnt else out_type[0],
        mesh=_sc_mesh(),
        compiler_params=pltpu.CompilerParams(use_tc_tiling_on_sc=False),
        scratch_types=scratch,
    )


def _gru_core(pa0, pa1, pc0, pc1, h, root, convb, wihT, bih, whhT, bhh):
    agg = pa0 + pa1
    cnt = pc0 + pc1
    mean = agg / jnp.maximum(cnt, 1.0)
    m = jax.nn.relu(
        mean + jnp.dot(h, root, preferred_element_type=F32) + convb)

    def gate(k):
        lo = k * _DC
        gi = (jnp.dot(m, wihT[:, lo:lo + _DC], preferred_element_type=F32)
              + bih[:, lo:lo + _DC])
        gh = (jnp.dot(h, whhT[:, lo:lo + _DC], preferred_element_type=F32)
              + bhh[:, lo:lo + _DC])
        return gi, gh

    i_r, h_r = gate(0)
    i_z, h_z = gate(1)
    i_n, h_n = gate(2)
    r = jax.nn.sigmoid(i_r + h_r)
    z = jax.nn.sigmoid(i_z + h_z)
    n = jnp.tanh(i_n + r * h_n)
    return (1.0 - z) * n + z * h


def _update_body(pa0_ref, pa1_ref, pc0_ref, pc1_ref, h_ref, root_ref,
                 convb_ref, wihT_ref, bih_ref, whhT_ref, bhh_ref, o_ref):
    o_ref[...] = _gru_core(
        pa0_ref[...], pa1_ref[...], pc0_ref[...], pc1_ref[...], h_ref[...],
        root_ref[...], convb_ref[...], wihT_ref[...], bih_ref[...],
        whhT_ref[...], bhh_ref[...])


def _update_final_body(pa0_ref, pa1_ref, pc0_ref, pc1_ref, h_ref, root_ref,
                       convb_ref, wihT_ref, bih_ref, whhT_ref, bhh_ref,
                       batch_ref, gattr_ref, w1a_ref, w1b_ref, bl1_ref,
                       w2_ref, bl2_ref, o_ref):
    h = _gru_core(
        pa0_ref[...], pa1_ref[...], pc0_ref[...], pc1_ref[...], h_ref[...],
        root_ref[...], convb_ref[...], wihT_ref[...], bih_ref[...],
        whhT_ref[...], bhh_ref[...])
    oh = (batch_ref[...] ==
          lax.broadcasted_iota(jnp.int32, (h.shape[0], _NG), 1)).astype(F32)
    gb = jnp.dot(oh, gattr_ref[...], preferred_element_type=F32)
    t = jax.nn.relu(
        jnp.dot(h, w1a_ref[...], preferred_element_type=F32)
        + jnp.dot(gb, w1b_ref[...], preferred_element_type=F32)
        + bl1_ref[...])
    o_ref[...] = (jnp.dot(t, w2_ref[...], preferred_element_type=F32)
                  + bl2_ref[...])


def _full(shape):
    return pl.BlockSpec(shape, lambda: (0,) * len(shape))


def _update(pagg, pcnt, h, root, convb, wihT, bih, whhT, bhh):
    args = (pagg[0], pagg[1], pcnt[0], pcnt[1], h, root,
            convb.reshape(1, -1), wihT, bih.reshape(1, -1), whhT,
            bhh.reshape(1, -1))
    return pl.pallas_call(
        _update_body,
        in_specs=[_full(a.shape) for a in args],
        out_specs=_full((_N, _DC)),
        out_shape=jax.ShapeDtypeStruct((_N, _DC), F32),
    )(*args)


def _update_final(pagg, pcnt, h, root, convb, wihT, bih, whhT, bhh,
                  batch, gattr, w1, bl1, w2, bl2):
    args = (pagg[0], pagg[1], pcnt[0], pcnt[1], h, root,
            convb.reshape(1, -1), wihT, bih.reshape(1, -1), whhT,
            bhh.reshape(1, -1), batch.reshape(-1, 1), gattr,
            w1[:_DC], w1[_DC:], bl1.reshape(1, -1), w2, bl2.reshape(1, -1))
    return pl.pallas_call(
        _update_final_body,
        in_specs=[_full(a.shape) for a in args],
        out_specs=_full((_N, w2.shape[1])),
        out_shape=jax.ShapeDtypeStruct((_N, w2.shape[1]), F32),
    )(*args)


def kernel(x, edge_index, edge_attr, graph_attr, batch, W0, b0, A1, bn1, A2,
           bn2, root, convb, Wih, bih, Whh, bhh, W1, bl1, W2, bl2):
    src2d = edge_index[0].reshape(_E // _CH, _CH)
    dst2d = edge_index[1].reshape(_E // _CH, _CH)
    wihT = Wih.T
    whhT = Whh.T
    ea_t = edge_attr.T

    h = _projection(x, W0, b0)

    g = _sc_gather()(h, src2d)
    msg = _messages(ea_t, g, A1, bn1, A2, bn2)
    pagg, pcnt = _sc_scatter(True)(msg, dst2d)
    h = _update(pagg, pcnt, h, root, convb, wihT, bih, whhT, bhh)

    g = _sc_gather()(h, src2d)
    msg = _messages(ea_t, g, A1, bn1, A2, bn2)
    pagg = _sc_scatter(False)(msg, dst2d)
    return _update_final(pagg, pcnt, h, root, convb, wihT, bih, whhT,
                         bhh, batch, graph_attr, W1, bl1, W2, bl2)
